# baseline (speedup 1.0000x reference)
import functools

import jax
import jax.numpy as jnp
from jax import lax
from jax.experimental import pallas as pl
from jax.experimental.pallas import tpu as pltpu
import jax.experimental.pallas.tpu_sc as plsc

N = 10000
NPAD = 10240
E = 320000
D_IN = 128
HID = 64
HEADS = 8
G = 64
CLASSES = 10

NC = 2
NS = 16
NW = NC * NS
EPW = E // NW
C = 400
NCHUNK = EPW // C
EPW2 = E // NS
NCHUNK2 = EPW2 // C
RPS = NPAD // NS
NEG = 0.2
GBYTES = C * HID * 4 + C * HEADS * 4
SBYTES = C * HID * 4


def _t1_body(x_ref, w1_ref, am_ref, h1_ref, asd_ref):
    h1 = jnp.dot(x_ref[...], w1_ref[...], preferred_element_type=jnp.float32)
    h1_ref[...] = h1
    asd_ref[...] = jnp.dot(h1, am_ref[...], preferred_element_type=jnp.float32)


_t1 = pl.pallas_call(
    _t1_body,
    grid=(10,),
    in_specs=[
        pl.BlockSpec((N // 10, D_IN), lambda i: (i, 0)),
        pl.BlockSpec((D_IN, HEADS * HID), lambda i: (0, 0)),
        pl.BlockSpec((HEADS * HID, 2 * HEADS), lambda i: (0, 0)),
    ],
    out_specs=[
        pl.BlockSpec((N // 10, HEADS * HID), lambda i: (i, 0)),
        pl.BlockSpec((N // 10, 2 * HEADS), lambda i: (i, 0)),
    ],
    out_shape=[
        jax.ShapeDtypeStruct((N, HEADS * HID), jnp.float32),
        jax.ShapeDtypeStruct((N, 2 * HEADS), jnp.float32),
    ],
)


def _edge_multiply(msgb, wvb, wcol, g_unused=None):
    def octet(ii, carry):
        for u in range(8):
            i = ii * 8 + u
            ic = jnp.full((16,), i, jnp.int32)
            wb = plsc.load_gather(wvb, [ic, jnp.full((16,), wcol, jnp.int32)])
            for q in range(HID // 16):
                msgb[i, pl.ds(q * 16, 16)] = msgb[i, pl.ds(q * 16, 16)] * wb
        return carry
    lax.fori_loop(0, C // 8, octet, 0)


def _sc1a_body(srcr_hbm, dstr_hbm, asd_hbm, z8_hbm,
               w_hbm, den_hbm,
               src_all, dst_all, a_src, a_dst, wrow, dacc, gsem):
    c = lax.axis_index("c")
    s = lax.axis_index("s")
    wid = c * NS + s
    ebase = wid * EPW
    r0 = s * RPS
    iota = lax.iota(jnp.int32, 16)

    pltpu.sync_copy(srcr_hbm.at[wid], src_all)
    pltpu.sync_copy(dstr_hbm.at[wid], dst_all)
    pltpu.sync_copy(z8_hbm.at[pl.ds(r0, RPS)], dacc.at[pl.ds(r0, RPS)])
    plsc.subcore_barrier()

    def chunk_a(j, carry):
        eb = ebase + j * C
        da = pltpu.async_copy(asd_hbm.at[src_all.at[j]], a_src, gsem)
        db = pltpu.async_copy(asd_hbm.at[dst_all.at[j]], a_dst, gsem)
        da.wait()
        db.wait()

        def tile(t, carry2):
            rows = iota + t * 16
            for h in range(HEADS):
                hc = jnp.full((16,), h, jnp.int32)
                sv = plsc.load_gather(a_src, [rows, hc])
                dv = plsc.load_gather(a_dst, [rows, hc + HEADS])
                e = sv + dv
                e = jnp.where(e > 0.0, e, NEG * e)
                wv = jnp.exp(e)
                plsc.store_scatter(wrow, [rows, hc], wv)
            return carry2

        lax.fori_loop(0, C // 16, tile, 0)
        pltpu.sync_copy(wrow, w_hbm.at[pl.ds(eb, C)])
        pltpu.sync_copy(wrow, dacc.at[dst_all.at[j]], add=True)
        return carry

    lax.fori_loop(0, NCHUNK, chunk_a, 0)
    plsc.subcore_barrier()
    pltpu.sync_copy(dacc.at[pl.ds(r0, RPS)], den_hbm.at[c, pl.ds(r0, RPS)])


_sc1a = functools.partial(
    pl.kernel,
    out_type=(
        jax.ShapeDtypeStruct((E, HEADS), jnp.float32),
        jax.ShapeDtypeStruct((NC, NPAD, HEADS), jnp.float32),
    ),
    mesh=plsc.VectorSubcoreMesh(core_axis_name="c", subcore_axis_name="s"),
    compiler_params=pltpu.CompilerParams(use_tc_tiling_on_sc=False, needs_layout_passes=False),
    scratch_types=[
        pltpu.VMEM((NCHUNK, C), jnp.int32),
        pltpu.VMEM((NCHUNK, C), jnp.int32),
        pltpu.VMEM((C, 16), jnp.float32),
        pltpu.VMEM((C, 16), jnp.float32),
        pltpu.VMEM((C, HEADS), jnp.float32),
        pltpu.VMEM_SHARED((NPAD, HEADS), jnp.float32),
        pltpu.SemaphoreType.DMA,
    ],
)(_sc1a_body)


def _sc1b_body(srcr_hbm, dstr_hbm, w_hbm, h1f_hbm, z64_hbm, z8_hbm,
               num_hbm,
               src_all, dst_all, msg0, msg1, wv0, wv1, acc,
               gsem0, gsem1, ssem0, ssem1):
    c = lax.axis_index("c")
    s = lax.axis_index("s")
    r0 = s * RPS

    def one_pass(g_local, carry0):
        gidx = 4 * c + g_local
        plsc.subcore_barrier()
        pltpu.sync_copy(z64_hbm.at[pl.ds(r0, RPS)], acc.at[pl.ds(r0, RPS)])
        plsc.subcore_barrier()

        def issue_g(j, ebase_h, msgb, wvb, sem):
            pltpu.async_copy(h1f_hbm.at[src_all.at[j]], msgb, sem)
            pltpu.async_copy(w_hbm.at[pl.ds(ebase_h + j * C, C)], wvb, sem)

        def issue_s(j, msgb, sem):
            pltpu.async_copy(msgb, acc.at[dst_all.at[j]], sem, add=True)

        def wait_g(msgb, wvb, sem):
            pltpu.make_async_copy(z64_hbm.at[pl.ds(0, C)], msgb, sem).wait()
            pltpu.make_async_copy(z8_hbm.at[pl.ds(0, C)], wvb, sem).wait()

        def wait_s(msgb, sem):
            pltpu.make_async_copy(z64_hbm.at[pl.ds(0, C)], msgb, sem).wait()

        def half_pass(hf, carryh, gidx=gidx):
            ebase_h = (s * 2 + hf) * EPW
            pltpu.sync_copy(srcr_hbm.at[s, hf], src_all)
            pltpu.sync_copy(dstr_hbm.at[s, hf], dst_all)

            def sh(t, carry):
                a = t // (C // 16)
                b = (t % (C // 16)) * 16
                src_all[a, pl.ds(b, 16)] = src_all[a, pl.ds(b, 16)] + gidx * N
                return carry
            carryh = lax.fori_loop(0, NCHUNK * (C // 16), sh, carryh)

            issue_g(0, ebase_h, msg0, wv0, gsem0)
            wait_g(msg0, wv0, gsem0)
            issue_g(1, ebase_h, msg1, wv1, gsem1)
            _edge_multiply(msg0, wv0, gidx)
            issue_s(0, msg0, ssem0)
            wait_g(msg1, wv1, gsem1)
            wait_s(msg0, ssem0)
            issue_g(2, ebase_h, msg0, wv0, gsem0)
            _edge_multiply(msg1, wv1, gidx)
            issue_s(1, msg1, ssem1)

            def pair(k, carry):
                j0 = 2 * k + 2
                wait_g(msg0, wv0, gsem0)
                wait_s(msg1, ssem1)
                issue_g(j0 + 1, ebase_h, msg1, wv1, gsem1)
                _edge_multiply(msg0, wv0, gidx)
                issue_s(j0, msg0, ssem0)
                wait_g(msg1, wv1, gsem1)
                wait_s(msg0, ssem0)
                issue_g(j0 + 2, ebase_h, msg0, wv0, gsem0)
                _edge_multiply(msg1, wv1, gidx)
                issue_s(j0 + 1, msg1, ssem1)
                return carry

            carryh = lax.fori_loop(0, (NCHUNK - 3) // 2, pair, carryh)
            wait_g(msg0, wv0, gsem0)
            wait_s(msg1, ssem1)
            _edge_multiply(msg0, wv0, gidx)
            pltpu.sync_copy(msg0, acc.at[dst_all.at[NCHUNK - 1]], add=True)
            return carryh

        carry0 = lax.fori_loop(0, 2, half_pass, carry0)
        plsc.subcore_barrier()
        pltpu.sync_copy(acc.at[pl.ds(r0, RPS)], num_hbm.at[gidx, pl.ds(r0, RPS)])
        return carry0

    lax.fori_loop(0, 4, one_pass, 0)


_sc1b = functools.partial(
    pl.kernel,
    out_type=(
        jax.ShapeDtypeStruct((HEADS, NPAD, HID), jnp.float32),
    ),
    mesh=plsc.VectorSubcoreMesh(core_axis_name="c", subcore_axis_name="s"),
    compiler_params=pltpu.CompilerParams(use_tc_tiling_on_sc=False, needs_layout_passes=False),
    scratch_types=[
        pltpu.VMEM((NCHUNK, C), jnp.int32),
        pltpu.VMEM((NCHUNK, C), jnp.int32),
        pltpu.VMEM((C, HID), jnp.float32),
        pltpu.VMEM((C, HID), jnp.float32),
        pltpu.VMEM((C, HEADS), jnp.float32),
        pltpu.VMEM((C, HEADS), jnp.float32),
        pltpu.VMEM_SHARED((NPAD, HID), jnp.float32),
        pltpu.SemaphoreType.DMA,
        pltpu.SemaphoreType.DMA,
        pltpu.SemaphoreType.DMA,
        pltpu.SemaphoreType.DMA,
    ],
)(_sc1b_body)


def _t2_body(num_ref, den_ref, b1_ref, w2_ref, am2_ref, dmat_ref, h2_ref, asd2_ref):
    num = num_ref[...]
    den = den_ref[0] + den_ref[1]
    rep = jnp.dot(den, dmat_ref[...], preferred_element_type=jnp.float32)
    h = jnp.maximum(num / (rep + 1e-16) + b1_ref[...], 0.0)
    h2 = jnp.dot(h, w2_ref[...], preferred_element_type=jnp.float32)
    h2_ref[...] = h2
    asd2_ref[...] = jnp.dot(h2, am2_ref[...], preferred_element_type=jnp.float32)


_t2 = pl.pallas_call(
    _t2_body,
    grid=(10,),
    in_specs=[
        pl.BlockSpec((NPAD // 10, HEADS * HID), lambda i: (i, 0)),
        pl.BlockSpec((NC, NPAD // 10, HEADS), lambda i: (0, i, 0)),
        pl.BlockSpec((1, HEADS * HID), lambda i: (0, 0)),
        pl.BlockSpec((HEADS * HID, HID), lambda i: (0, 0)),
        pl.BlockSpec((HID, 2 * HEADS), lambda i: (0, 0)),
        pl.BlockSpec((HEADS, HEADS * HID), lambda i: (0, 0)),
    ],
    out_specs=[
        pl.BlockSpec((NPAD // 10, HID), lambda i: (i, 0)),
        pl.BlockSpec((NPAD // 10, 2 * HEADS), lambda i: (i, 0)),
    ],
    out_shape=[
        jax.ShapeDtypeStruct((NPAD, HID), jnp.float32),
        jax.ShapeDtypeStruct((NPAD, 2 * HEADS), jnp.float32),
    ],
)


def _sc2_body(srcr_hbm, dstr_hbm, asd_hbm, h2_hbm, z64_hbm, z8_hbm,
              num_hbm, den_hbm,
              src_all, dst_all, a_src, a_dst, wrow,
              msg0, msg1, acc, dacc, gsem0, gsem1):
    c = lax.axis_index("c")
    s = lax.axis_index("s")
    wid = c * NS + s
    ebase = wid * EPW
    r0 = s * RPS
    iota = lax.iota(jnp.int32, 16)

    pltpu.sync_copy(srcr_hbm.at[wid], src_all)
    pltpu.sync_copy(dstr_hbm.at[wid], dst_all)
    pltpu.sync_copy(z8_hbm.at[pl.ds(0, C)], wrow)
    pltpu.sync_copy(z64_hbm.at[pl.ds(r0, RPS)], acc.at[pl.ds(r0, RPS)])
    pltpu.sync_copy(z8_hbm.at[pl.ds(r0, RPS)], dacc.at[pl.ds(r0, RPS)])
    plsc.subcore_barrier()

    def consume(j, msgb):
        dm = pltpu.async_copy(h2_hbm.at[src_all.at[j]], msgb, gsem0)
        da = pltpu.async_copy(asd_hbm.at[src_all.at[j]], a_src, gsem1)
        db = pltpu.async_copy(asd_hbm.at[dst_all.at[j]], a_dst, gsem1)
        da.wait()
        db.wait()

        def tile(t, carry2):
            rows = iota + t * 16
            zc = jnp.full((16,), 0, jnp.int32)
            sv = plsc.load_gather(a_src, [rows, zc])
            dv = plsc.load_gather(a_dst, [rows, zc + HEADS])
            e = sv + dv
            e = jnp.where(e > 0.0, e, NEG * e)
            wv = jnp.exp(e)
            plsc.store_scatter(wrow, [rows, zc], wv)
            return carry2

        lax.fori_loop(0, C // 16, tile, 0)
        dm.wait()
        _edge_multiply(msgb, wrow, 0)
        pltpu.sync_copy(msgb, acc.at[dst_all.at[j]], add=True)
        pltpu.sync_copy(wrow, dacc.at[dst_all.at[j]], add=True)

    def chunk_loop(j, carry):
        consume(j, msg0)
        return carry

    lax.fori_loop(0, NCHUNK, chunk_loop, 0)

    plsc.subcore_barrier()
    pltpu.sync_copy(acc.at[pl.ds(r0, RPS)], num_hbm.at[c, pl.ds(r0, RPS)])
    pltpu.sync_copy(dacc.at[pl.ds(r0, RPS)], den_hbm.at[c, pl.ds(r0, RPS)])


_sc2 = functools.partial(
    pl.kernel,
    out_type=(
        jax.ShapeDtypeStruct((NC, NPAD, HID), jnp.float32),
        jax.ShapeDtypeStruct((NC, NPAD, HEADS), jnp.float32),
    ),
    mesh=plsc.VectorSubcoreMesh(core_axis_name="c", subcore_axis_name="s"),
    compiler_params=pltpu.CompilerParams(use_tc_tiling_on_sc=False, needs_layout_passes=False),
    scratch_types=[
        pltpu.VMEM((NCHUNK, C), jnp.int32),
        pltpu.VMEM((NCHUNK, C), jnp.int32),
        pltpu.VMEM((C, 16), jnp.float32),
        pltpu.VMEM((C, 16), jnp.float32),
        pltpu.VMEM((C, HEADS), jnp.float32),
        pltpu.VMEM((C, HID), jnp.float32),
        pltpu.VMEM((C, HID), jnp.float32),
        pltpu.VMEM_SHARED((NPAD, HID), jnp.float32),
        pltpu.VMEM_SHARED((NPAD, HEADS), jnp.float32),
        pltpu.SemaphoreType.DMA,
        pltpu.SemaphoreType.DMA,
    ],
)(_sc2_body)


def _t3_body(num_ref, den_ref, batch_ref, b2_ref, fcw_ref, fcb_ref, out_ref):
    num = num_ref[0] + num_ref[1]
    den = (den_ref[0] + den_ref[1])[:, 0:1]
    h = num / (den + 1e-16) + b2_ref[...]
    gids = lax.broadcasted_iota(jnp.int32, (G, N), 0)
    oh = (gids == batch_ref[...]).astype(jnp.float32)
    gp = jax.lax.dot(oh, h)
    logits = jax.lax.dot(gp, fcw_ref[...]) + fcb_ref[...]
    m = jnp.max(logits, axis=1, keepdims=True)
    z = logits - m
    lse = jnp.log(jnp.sum(jnp.exp(z), axis=1, keepdims=True))
    out_ref[...] = z - lse


_t3 = pl.pallas_call(
    _t3_body,
    out_shape=jax.ShapeDtypeStruct((G, CLASSES), jnp.float32),
)


def kernel(x, edge_index, batch, W1, a_s1, a_d1, b1, W2, a_s2, a_d2, b2, fc_W, fc_b):
    f32 = jnp.float32
    src = edge_index[0].reshape(NW, NCHUNK, C)
    dst = edge_index[1].reshape(NW, NCHUNK, C)

    eye8 = jnp.eye(HEADS, dtype=f32)
    asm1 = (eye8[:, None, :] * a_s1[:, :, None]).reshape(HEADS * HID, HEADS)
    adm1 = (eye8[:, None, :] * a_d1[:, :, None]).reshape(HEADS * HID, HEADS)
    am1 = jnp.concatenate([asm1, adm1], axis=1)

    srcH = edge_index[0].reshape(NS, 2, NCHUNK, C)
    dstH = edge_index[1].reshape(NS, 2, NCHUNK, C)

    h1, asd1 = _t1(x, W1, am1)
    h1f = jnp.transpose(h1.reshape(N, HEADS, HID), (1, 0, 2)).reshape(HEADS * N, HID)
    z64 = jnp.zeros((NPAD, HID), f32)
    z8 = jnp.zeros((NPAD, HEADS), f32)

    w1e, den1p = _sc1a(src, dst, asd1, z8)
    (num1p,) = _sc1b(srcH, dstH, w1e, h1f, z64, z8)
    num1 = jnp.transpose(num1p, (1, 0, 2)).reshape(NPAD, HEADS * HID)

    am2 = jnp.concatenate(
        [a_s2.T, jnp.zeros((HID, 7), f32), a_d2.T, jnp.zeros((HID, 7), f32)], axis=1)
    dmat = jnp.repeat(eye8, HID, axis=1)

    h2, asd2 = _t2(num1, den1p, b1.reshape(1, HEADS * HID), W2, am2, dmat)

    num2p, den2p = _sc2(src, dst, asd2, h2, z64, z8)

    return _t3(num2p[:, :N], den2p[:, :N], batch.reshape(1, N).astype(jnp.int32),
               b2.reshape(1, HID), fc_W, fc_b.reshape(1, CLASSES))


# speedup vs baseline: 1.0046x; 1.0046x over previous
import functools

import jax
import jax.numpy as jnp
from jax import lax
from jax.experimental import pallas as pl
from jax.experimental.pallas import tpu as pltpu
import jax.experimental.pallas.tpu_sc as plsc

N = 10000
NPAD = 10240
E = 320000
D_IN = 128
HID = 64
HEADS = 8
G = 64
CLASSES = 10

NC = 2
NS = 16
NW = NC * NS
EPW = E // NW
C = 400
NCHUNK = EPW // C
EPW2 = E // NS
NCHUNK2 = EPW2 // C
RPS = NPAD // NS
NEG = 0.2
GBYTES = C * HID * 4 + C * HEADS * 4
SBYTES = C * HID * 4


def _t1_body(x_ref, w1_ref, am_ref, h1_ref, asd_ref):
    h1 = jnp.dot(x_ref[...], w1_ref[...], preferred_element_type=jnp.float32)
    h1_ref[...] = h1
    asd_ref[...] = jnp.dot(h1, am_ref[...], preferred_element_type=jnp.float32)


_t1 = pl.pallas_call(
    _t1_body,
    grid=(10,),
    in_specs=[
        pl.BlockSpec((N // 10, D_IN), lambda i: (i, 0)),
        pl.BlockSpec((D_IN, HEADS * HID), lambda i: (0, 0)),
        pl.BlockSpec((HEADS * HID, 2 * HEADS), lambda i: (0, 0)),
    ],
    out_specs=[
        pl.BlockSpec((N // 10, HEADS * HID), lambda i: (i, 0)),
        pl.BlockSpec((N // 10, 2 * HEADS), lambda i: (i, 0)),
    ],
    out_shape=[
        jax.ShapeDtypeStruct((N, HEADS * HID), jnp.float32),
        jax.ShapeDtypeStruct((N, 2 * HEADS), jnp.float32),
    ],
)


def _edge_multiply(msgb, wvb, wcol, g_unused=None):
    def quad(ii, carry):
        for u in range(4):
            i = ii * 4 + u
            ic = jnp.full((16,), i, jnp.int32)
            wb = plsc.load_gather(wvb, [ic, jnp.full((16,), wcol, jnp.int32)])
            for q in range(HID // 16):
                msgb[i, pl.ds(q * 16, 16)] = msgb[i, pl.ds(q * 16, 16)] * wb
        return carry
    lax.fori_loop(0, C // 4, quad, 0)


def _sc1a_body(srcr_hbm, dstr_hbm, asd_hbm, z8_hbm,
               w_hbm, den_hbm,
               src_all, dst_all, a_src, a_dst, wrow, dacc, gsem):
    c = lax.axis_index("c")
    s = lax.axis_index("s")
    wid = c * NS + s
    ebase = wid * EPW
    r0 = s * RPS
    iota = lax.iota(jnp.int32, 16)

    pltpu.sync_copy(srcr_hbm.at[wid], src_all)
    pltpu.sync_copy(dstr_hbm.at[wid], dst_all)
    pltpu.sync_copy(z8_hbm.at[pl.ds(r0, RPS)], dacc.at[pl.ds(r0, RPS)])
    plsc.subcore_barrier()

    def chunk_a(j, carry):
        eb = ebase + j * C
        da = pltpu.async_copy(asd_hbm.at[src_all.at[j]], a_src, gsem)
        db = pltpu.async_copy(asd_hbm.at[dst_all.at[j]], a_dst, gsem)
        da.wait()
        db.wait()

        def tile(t, carry2):
            rows = iota + t * 16
            for h in range(HEADS):
                hc = jnp.full((16,), h, jnp.int32)
                sv = plsc.load_gather(a_src, [rows, hc])
                dv = plsc.load_gather(a_dst, [rows, hc + HEADS])
                e = sv + dv
                e = jnp.where(e > 0.0, e, NEG * e)
                wv = jnp.exp(e)
                plsc.store_scatter(wrow, [rows, hc], wv)
            return carry2

        lax.fori_loop(0, C // 16, tile, 0)
        pltpu.sync_copy(wrow, w_hbm.at[pl.ds(eb, C)])
        pltpu.sync_copy(wrow, dacc.at[dst_all.at[j]], add=True)
        return carry

    lax.fori_loop(0, NCHUNK, chunk_a, 0)
    plsc.subcore_barrier()
    pltpu.sync_copy(dacc.at[pl.ds(r0, RPS)], den_hbm.at[c, pl.ds(r0, RPS)])


_sc1a = functools.partial(
    pl.kernel,
    out_type=(
        jax.ShapeDtypeStruct((E, HEADS), jnp.float32),
        jax.ShapeDtypeStruct((NC, NPAD, HEADS), jnp.float32),
    ),
    mesh=plsc.VectorSubcoreMesh(core_axis_name="c", subcore_axis_name="s"),
    compiler_params=pltpu.CompilerParams(use_tc_tiling_on_sc=False, needs_layout_passes=False),
    scratch_types=[
        pltpu.VMEM((NCHUNK, C), jnp.int32),
        pltpu.VMEM((NCHUNK, C), jnp.int32),
        pltpu.VMEM((C, 16), jnp.float32),
        pltpu.VMEM((C, 16), jnp.float32),
        pltpu.VMEM((C, HEADS), jnp.float32),
        pltpu.VMEM_SHARED((NPAD, HEADS), jnp.float32),
        pltpu.SemaphoreType.DMA,
    ],
)(_sc1a_body)


def _sc1b_body(srcr_hbm, dstr_hbm, w_hbm, h1f_hbm, z64_hbm, z8_hbm,
               num_hbm,
               src_all, dst_all, msg0, msg1, wv0, wv1, acc,
               gsem0, gsem1, ssem0, ssem1):
    c = lax.axis_index("c")
    s = lax.axis_index("s")
    r0 = s * RPS

    def one_pass(g_local, carry0):
        gidx = 4 * c + g_local
        plsc.subcore_barrier()
        pltpu.sync_copy(z64_hbm.at[pl.ds(r0, RPS)], acc.at[pl.ds(r0, RPS)])
        plsc.subcore_barrier()

        def issue_g(j, ebase_h, msgb, wvb, sem):
            pltpu.async_copy(h1f_hbm.at[src_all.at[j]], msgb, sem)
            pltpu.async_copy(w_hbm.at[pl.ds(ebase_h + j * C, C)], wvb, sem)

        def issue_s(j, msgb, sem):
            pltpu.async_copy(msgb, acc.at[dst_all.at[j]], sem, add=True)

        def wait_g(msgb, wvb, sem):
            pltpu.make_async_copy(z64_hbm.at[pl.ds(0, C)], msgb, sem).wait()
            pltpu.make_async_copy(z8_hbm.at[pl.ds(0, C)], wvb, sem).wait()

        def wait_s(msgb, sem):
            pltpu.make_async_copy(z64_hbm.at[pl.ds(0, C)], msgb, sem).wait()

        def half_pass(hf, carryh, gidx=gidx):
            ebase_h = (s * 2 + hf) * EPW
            pltpu.sync_copy(srcr_hbm.at[s, hf], src_all)
            pltpu.sync_copy(dstr_hbm.at[s, hf], dst_all)

            def sh(t, carry):
                a = t // (C // 16)
                b = (t % (C // 16)) * 16
                src_all[a, pl.ds(b, 16)] = src_all[a, pl.ds(b, 16)] + gidx * N
                return carry
            carryh = lax.fori_loop(0, NCHUNK * (C // 16), sh, carryh)

            issue_g(0, ebase_h, msg0, wv0, gsem0)
            wait_g(msg0, wv0, gsem0)
            issue_g(1, ebase_h, msg1, wv1, gsem1)
            _edge_multiply(msg0, wv0, gidx)
            issue_s(0, msg0, ssem0)
            wait_g(msg1, wv1, gsem1)
            wait_s(msg0, ssem0)
            issue_g(2, ebase_h, msg0, wv0, gsem0)
            _edge_multiply(msg1, wv1, gidx)
            issue_s(1, msg1, ssem1)

            def pair(k, carry):
                j0 = 2 * k + 2
                wait_g(msg0, wv0, gsem0)
                wait_s(msg1, ssem1)
                issue_g(j0 + 1, ebase_h, msg1, wv1, gsem1)
                _edge_multiply(msg0, wv0, gidx)
                issue_s(j0, msg0, ssem0)
                wait_g(msg1, wv1, gsem1)
                wait_s(msg0, ssem0)
                issue_g(j0 + 2, ebase_h, msg0, wv0, gsem0)
                _edge_multiply(msg1, wv1, gidx)
                issue_s(j0 + 1, msg1, ssem1)
                return carry

            carryh = lax.fori_loop(0, (NCHUNK - 3) // 2, pair, carryh)
            wait_g(msg0, wv0, gsem0)
            wait_s(msg1, ssem1)
            _edge_multiply(msg0, wv0, gidx)
            pltpu.sync_copy(msg0, acc.at[dst_all.at[NCHUNK - 1]], add=True)
            return carryh

        carry0 = lax.fori_loop(0, 2, half_pass, carry0)
        plsc.subcore_barrier()
        pltpu.sync_copy(acc.at[pl.ds(r0, RPS)], num_hbm.at[gidx, pl.ds(r0, RPS)])
        return carry0

    lax.fori_loop(0, 4, one_pass, 0)


_sc1b = functools.partial(
    pl.kernel,
    out_type=(
        jax.ShapeDtypeStruct((HEADS, NPAD, HID), jnp.float32),
    ),
    mesh=plsc.VectorSubcoreMesh(core_axis_name="c", subcore_axis_name="s"),
    compiler_params=pltpu.CompilerParams(use_tc_tiling_on_sc=False, needs_layout_passes=False),
    scratch_types=[
        pltpu.VMEM((NCHUNK, C), jnp.int32),
        pltpu.VMEM((NCHUNK, C), jnp.int32),
        pltpu.VMEM((C, HID), jnp.float32),
        pltpu.VMEM((C, HID), jnp.float32),
        pltpu.VMEM((C, HEADS), jnp.float32),
        pltpu.VMEM((C, HEADS), jnp.float32),
        pltpu.VMEM_SHARED((NPAD, HID), jnp.float32),
        pltpu.SemaphoreType.DMA,
        pltpu.SemaphoreType.DMA,
        pltpu.SemaphoreType.DMA,
        pltpu.SemaphoreType.DMA,
    ],
)(_sc1b_body)


def _t2_body(num_ref, den_ref, b1_ref, w2_ref, am2_ref, dmat_ref, h2_ref, asd2_ref):
    num = num_ref[...]
    den = den_ref[0] + den_ref[1]
    rep = jnp.dot(den, dmat_ref[...], preferred_element_type=jnp.float32)
    h = jnp.maximum(num / (rep + 1e-16) + b1_ref[...], 0.0)
    h2 = jnp.dot(h, w2_ref[...], preferred_element_type=jnp.float32)
    h2_ref[...] = h2
    asd2_ref[...] = jnp.dot(h2, am2_ref[...], preferred_element_type=jnp.float32)


_t2 = pl.pallas_call(
    _t2_body,
    grid=(10,),
    in_specs=[
        pl.BlockSpec((NPAD // 10, HEADS * HID), lambda i: (i, 0)),
        pl.BlockSpec((NC, NPAD // 10, HEADS), lambda i: (0, i, 0)),
        pl.BlockSpec((1, HEADS * HID), lambda i: (0, 0)),
        pl.BlockSpec((HEADS * HID, HID), lambda i: (0, 0)),
        pl.BlockSpec((HID, 2 * HEADS), lambda i: (0, 0)),
        pl.BlockSpec((HEADS, HEADS * HID), lambda i: (0, 0)),
    ],
    out_specs=[
        pl.BlockSpec((NPAD // 10, HID), lambda i: (i, 0)),
        pl.BlockSpec((NPAD // 10, 2 * HEADS), lambda i: (i, 0)),
    ],
    out_shape=[
        jax.ShapeDtypeStruct((NPAD, HID), jnp.float32),
        jax.ShapeDtypeStruct((NPAD, 2 * HEADS), jnp.float32),
    ],
)


def _sc2_body(srcr_hbm, dstr_hbm, asd_hbm, h2_hbm, z64_hbm, z8_hbm,
              num_hbm, den_hbm,
              src_all, dst_all, a_src, a_dst, wrow,
              msg0, msg1, acc, dacc, gsem0, gsem1):
    c = lax.axis_index("c")
    s = lax.axis_index("s")
    wid = c * NS + s
    ebase = wid * EPW
    r0 = s * RPS
    iota = lax.iota(jnp.int32, 16)

    pltpu.sync_copy(srcr_hbm.at[wid], src_all)
    pltpu.sync_copy(dstr_hbm.at[wid], dst_all)
    pltpu.sync_copy(z8_hbm.at[pl.ds(0, C)], wrow)
    pltpu.sync_copy(z64_hbm.at[pl.ds(r0, RPS)], acc.at[pl.ds(r0, RPS)])
    pltpu.sync_copy(z8_hbm.at[pl.ds(r0, RPS)], dacc.at[pl.ds(r0, RPS)])
    plsc.subcore_barrier()

    def consume(j, msgb):
        dm = pltpu.async_copy(h2_hbm.at[src_all.at[j]], msgb, gsem0)
        da = pltpu.async_copy(asd_hbm.at[src_all.at[j]], a_src, gsem1)
        db = pltpu.async_copy(asd_hbm.at[dst_all.at[j]], a_dst, gsem1)
        da.wait()
        db.wait()

        def tile(t, carry2):
            rows = iota + t * 16
            zc = jnp.full((16,), 0, jnp.int32)
            sv = plsc.load_gather(a_src, [rows, zc])
            dv = plsc.load_gather(a_dst, [rows, zc + HEADS])
            e = sv + dv
            e = jnp.where(e > 0.0, e, NEG * e)
            wv = jnp.exp(e)
            plsc.store_scatter(wrow, [rows, zc], wv)
            return carry2

        lax.fori_loop(0, C // 16, tile, 0)
        dm.wait()
        _edge_multiply(msgb, wrow, 0)
        pltpu.sync_copy(msgb, acc.at[dst_all.at[j]], add=True)
        pltpu.sync_copy(wrow, dacc.at[dst_all.at[j]], add=True)

    def chunk_loop(j, carry):
        consume(j, msg0)
        return carry

    lax.fori_loop(0, NCHUNK, chunk_loop, 0)

    plsc.subcore_barrier()
    pltpu.sync_copy(acc.at[pl.ds(r0, RPS)], num_hbm.at[c, pl.ds(r0, RPS)])
    pltpu.sync_copy(dacc.at[pl.ds(r0, RPS)], den_hbm.at[c, pl.ds(r0, RPS)])


_sc2 = functools.partial(
    pl.kernel,
    out_type=(
        jax.ShapeDtypeStruct((NC, NPAD, HID), jnp.float32),
        jax.ShapeDtypeStruct((NC, NPAD, HEADS), jnp.float32),
    ),
    mesh=plsc.VectorSubcoreMesh(core_axis_name="c", subcore_axis_name="s"),
    compiler_params=pltpu.CompilerParams(use_tc_tiling_on_sc=False, needs_layout_passes=False),
    scratch_types=[
        pltpu.VMEM((NCHUNK, C), jnp.int32),
        pltpu.VMEM((NCHUNK, C), jnp.int32),
        pltpu.VMEM((C, 16), jnp.float32),
        pltpu.VMEM((C, 16), jnp.float32),
        pltpu.VMEM((C, HEADS), jnp.float32),
        pltpu.VMEM((C, HID), jnp.float32),
        pltpu.VMEM((C, HID), jnp.float32),
        pltpu.VMEM_SHARED((NPAD, HID), jnp.float32),
        pltpu.VMEM_SHARED((NPAD, HEADS), jnp.float32),
        pltpu.SemaphoreType.DMA,
        pltpu.SemaphoreType.DMA,
    ],
)(_sc2_body)


def _t3_body(num_ref, den_ref, batch_ref, b2_ref, fcw_ref, fcb_ref, out_ref):
    num = num_ref[0] + num_ref[1]
    den = (den_ref[0] + den_ref[1])[:, 0:1]
    h = num / (den + 1e-16) + b2_ref[...]
    gids = lax.broadcasted_iota(jnp.int32, (G, N), 0)
    oh = (gids == batch_ref[...]).astype(jnp.float32)
    gp = jax.lax.dot(oh, h)
    logits = jax.lax.dot(gp, fcw_ref[...]) + fcb_ref[...]
    m = jnp.max(logits, axis=1, keepdims=True)
    z = logits - m
    lse = jnp.log(jnp.sum(jnp.exp(z), axis=1, keepdims=True))
    out_ref[...] = z - lse


_t3 = pl.pallas_call(
    _t3_body,
    out_shape=jax.ShapeDtypeStruct((G, CLASSES), jnp.float32),
)


def kernel(x, edge_index, batch, W1, a_s1, a_d1, b1, W2, a_s2, a_d2, b2, fc_W, fc_b):
    f32 = jnp.float32
    src = edge_index[0].reshape(NW, NCHUNK, C)
    dst = edge_index[1].reshape(NW, NCHUNK, C)

    eye8 = jnp.eye(HEADS, dtype=f32)
    asm1 = (eye8[:, None, :] * a_s1[:, :, None]).reshape(HEADS * HID, HEADS)
    adm1 = (eye8[:, None, :] * a_d1[:, :, None]).reshape(HEADS * HID, HEADS)
    am1 = jnp.concatenate([asm1, adm1], axis=1)

    srcH = edge_index[0].reshape(NS, 2, NCHUNK, C)
    dstH = edge_index[1].reshape(NS, 2, NCHUNK, C)

    h1, asd1 = _t1(x, W1, am1)
    h1f = jnp.transpose(h1.reshape(N, HEADS, HID), (1, 0, 2)).reshape(HEADS * N, HID)
    z64 = jnp.zeros((NPAD, HID), f32)
    z8 = jnp.zeros((NPAD, HEADS), f32)

    w1e, den1p = _sc1a(src, dst, asd1, z8)
    (num1p,) = _sc1b(srcH, dstH, w1e, h1f, z64, z8)
    num1 = jnp.transpose(num1p, (1, 0, 2)).reshape(NPAD, HEADS * HID)

    am2 = jnp.concatenate(
        [a_s2.T, jnp.zeros((HID, 7), f32), a_d2.T, jnp.zeros((HID, 7), f32)], axis=1)
    dmat = jnp.repeat(eye8, HID, axis=1)

    h2, asd2 = _t2(num1, den1p, b1.reshape(1, HEADS * HID), W2, am2, dmat)

    num2p, den2p = _sc2(src, dst, asd2, h2, z64, z8)

    return _t3(num2p[:, :N], den2p[:, :N], batch.reshape(1, N).astype(jnp.int32),
               b2.reshape(1, HID), fc_W, fc_b.reshape(1, CLASSES))
